# Initial kernel scaffold; baseline (speedup 1.0000x reference)
import jax
import jax.numpy as jnp
from jax.experimental import pallas as pl


def kernel(rays, intersection, W1, W2, Wsig, Wsem, Wfeat, Wr1, Wr2):
    raise NotImplementedError("write your pallas kernel here")


# speedup vs baseline: 6.3420x; 6.3420x over previous
import numpy as np
import jax
import jax.numpy as jnp
from jax.experimental import pallas as pl

_R = 1024
_NB = 8
_SP = 8
_NZ = _NB * _SP
_CASC = 128
_S = _CASC + 2 * _NB
_NC = 50
_DIST = 100.0
_FR_POS = 10
_FR_DIR = 4
_WH = 128
_MERGE_PAIRS = [(39, 41), (27, 26), (28, 26), (29, 26), (30, 26), (31, 26),
                (9, 8), (35, 13)]

_RB = 32


def _pdf_consts():
    w = np.full((_NZ - 1,), 1.0, np.float32) + np.float32(1e-5)
    pdf = (w / w.sum(dtype=np.float32)).astype(np.float32)
    cdf = np.concatenate([np.zeros((1,), np.float32),
                          np.cumsum(pdf, dtype=np.float32).astype(np.float32)])
    u = np.linspace(0.0, 1.0, _CASC).astype(np.float32)
    inds = np.sum((u[:, None] >= cdf[None, :]).astype(np.int32), axis=-1)
    below = np.clip(inds - 1, 0, _NZ - 1).astype(np.int32)
    above = np.clip(inds, 0, _NZ - 1).astype(np.int32)
    denom = cdf[above] - cdf[below]
    denom = np.where(denom < 1e-5, np.float32(1.0), denom).astype(np.float32)
    t = ((u - cdf[below]) / denom).astype(np.float32)
    return below, above, t

_BELOW, _ABOVE, _TFRAC = _pdf_consts()
_TVEC = np.tile(np.linspace(0.0, 1.0, _SP).astype(np.float32), _NB)
_FREQS_POS = (2.0 ** np.arange(_FR_POS)).astype(np.float32)
_FREQS_DIR = (2.0 ** np.arange(_FR_DIR)).astype(np.float32)

def _enc_perm(L):
    return np.array(
        [0, 1, 2]
        + [3 + 6 * l + c for l in range(L) for c in range(3)]
        + [6 + 6 * l + c for l in range(L) for c in range(3)], np.int32)

_PERM_POS = _enc_perm(_FR_POS)
_PERM_DIR = _enc_perm(_FR_DIR)


def _ranks(vals):
    n = vals.shape[-1]
    ei = jax.lax.broadcasted_iota(jnp.int32, (n, n), 0)
    fi = jax.lax.broadcasted_iota(jnp.int32, (n, n), 1)
    a = vals[:, :, None]
    b = vals[:, None, :]
    less = (b < a) | ((b == a) & (fi < ei)[None])
    return jnp.sum(less.astype(jnp.int32), axis=2)


def _body(rays_ref, inter_ref, tvec_ref, below_ref, above_ref, tfrac_ref,
          w1_ref, w2_ref, wcat_ref, wr1f_ref, wr1d_ref,
          wr2_ref, rgb_ref, depth_ref, acc_ref, sem_ref, oh_ref):
    rays = rays_ref[...]
    origin = rays[:, 0:3]
    dvec = rays[:, 3:6]
    inter = inter_ref[...]
    near = inter[:, :, 0]
    far = inter[:, :, 1]
    cls = inter[:, :, 3].astype(jnp.int32)
    scale = jnp.sqrt(jnp.sum(dvec * dvec, axis=-1, keepdims=True))

    tvec = tvec_ref[...]
    grp = jax.lax.broadcasted_iota(jnp.int32, (1, _NZ), 1) // _SP
    near64 = jnp.zeros((_RB, _NZ), jnp.float32)
    far64 = jnp.zeros((_RB, _NZ), jnp.float32)
    for b in range(_NB):
        near64 = jnp.where(grp == b, near[:, b:b + 1], near64)
        far64 = jnp.where(grp == b, far[:, b:b + 1], far64)
    zf = near64 * (1.0 - tvec) + far64 * tvec

    rank64 = _ranks(zf)
    below = below_ref[...]
    above = above_ref[...]
    tfrac = tfrac_ref[...]
    m0 = rank64[:, :, None] == below[:, None, :]
    m1 = rank64[:, :, None] == above[:, None, :]
    bg0 = jnp.sum(jnp.where(m0, zf[:, :, None], 0.0), axis=1)
    bg1 = jnp.sum(jnp.where(m1, zf[:, :, None], 0.0), axis=1)
    zv128 = bg0 + tfrac * (bg1 - bg0)

    zb = jnp.concatenate([near - 1e-5, far + 1e-5], axis=1)
    zcat = jnp.concatenate([zv128, zb], axis=1)
    rank144 = _ranks(zcat)
    k144 = jax.lax.broadcasted_iota(jnp.int32, (_S, _S), 1)
    msort = rank144[:, :, None] == k144[None]
    zv = jnp.sum(jnp.where(msort, zcat[:, :, None], 0.0), axis=1)

    pts = dvec[:, None, :] * zv[:, :, None] / scale[:, :, None]
    xyz = (origin[:, None, :] + pts) / _DIST
    x = xyz.reshape(_RB * _S, 3)
    xall = jnp.concatenate([x * f for f in _FREQS_POS], axis=1)
    pe = jnp.concatenate(
        [x, jnp.sin(xall), jnp.cos(xall),
         jnp.zeros((_RB * _S, 1), jnp.float32)], axis=1)
    dall = jnp.concatenate([dvec * f for f in _FREQS_DIR], axis=1)
    dpe = jnp.concatenate(
        [dvec, jnp.sin(dall), jnp.cos(dall),
         jnp.zeros((_RB, 5), jnp.float32)], axis=1)

    h = jnp.maximum(jnp.dot(pe, w1_ref[...],
                            preferred_element_type=jnp.float32), 0.0)
    h = jnp.maximum(jnp.dot(h, w2_ref[...],
                            preferred_element_type=jnp.float32), 0.0)
    hcat = jnp.dot(h, wcat_ref[...], preferred_element_type=jnp.float32)
    feat = hcat[:, :_WH]
    ddot = jnp.dot(dpe, wr1d_ref[...],
                   preferred_element_type=jnp.float32)
    ddot_b = jnp.broadcast_to(ddot[:, None, :], (_RB, _S, 64)).reshape(
        _RB * _S, 64)
    hr = jnp.maximum(jnp.dot(feat, wr1f_ref[...],
                             preferred_element_type=jnp.float32) + ddot_b, 0.0)
    rgbl = jnp.dot(hr, wr2_ref[...], preferred_element_type=jnp.float32)
    rgb = jax.nn.sigmoid(rgbl)

    sem_r = hcat[:, _WH:_WH + _NC].reshape(_RB, _S, _NC)
    sigma3 = hcat[:, _WH + _NC:_WH + _NC + 1].reshape(_RB, _S, 1)
    sigma_r = jnp.sum(sigma3, axis=2)
    rgb_r = rgb.reshape(_RB, _S, 3)

    inside = ((zv[:, :, None] > near[:, None, :]) &
              (zv[:, :, None] < far[:, None, :]))
    dfar = zv[:, :, None] - far[:, None, :]
    dnear = near[:, None, :] - zv[:, :, None]
    bound = (((dfar < 1e-3) & (dfar > 0)) |
             ((dnear > 0) & (dnear < 1e-3)))
    bound_any = jnp.sum(bound.astype(jnp.int32), axis=2) > 0

    c50 = jax.lax.broadcasted_iota(jnp.int32, (_RB, _NB, _NC), 2)
    ohc = (cls[:, :, None] == c50).astype(jnp.float32)
    inside_f = inside.astype(jnp.float32)
    onehot = jnp.zeros((_RB, _S, _NC), jnp.float32)
    for b in range(_NB):
        onehot = onehot + inside_f[:, :, b:b + 1] * ohc[:, b:b + 1, :]
    onehot = jnp.minimum(onehot, 1.0)

    ssum = jnp.sum(onehot, axis=2)
    ssum3 = jnp.sum(onehot, axis=2, keepdims=True)
    mask_bbox = (zv < _DIST) & (ssum == 0.0)
    zv3 = zv[:, :, None]
    mask_bbox3 = (zv3 < _DIST) & (ssum3 == 0.0)
    mask_bg3 = (zv3 > _DIST) & (ssum3 == 0.0)
    col = jax.lax.broadcasted_iota(jnp.int32, (_RB, _S, _NC), 2)
    onehot = jnp.where((col == 0) & mask_bbox3, 1.0, onehot)
    onehot = jnp.where((col == 23) & mask_bg3, 1.0, onehot)
    for i, tgt in _MERGE_PAIRS:
        m = onehot[:, :, i:i + 1] == 1.0
        onehot = jnp.where((col == i) & m, 0.0, onehot)
        onehot = jnp.where((col == tgt) & m, 1.0, onehot)

    sigma_r = jnp.where(mask_bbox | bound_any, 0.0, sigma_r)
    zvs = zv / scale
    dists = jnp.concatenate(
        [zvs[:, 1:] - zvs[:, :-1],
         jnp.full((_RB, 1), 1e10, jnp.float32)], axis=1)
    dists = dists * scale
    alpha = 1.0 - jnp.exp(-jnp.maximum(sigma_r, 0.0) * dists)
    v = 1.0 - alpha + 1e-10
    p = jnp.concatenate([jnp.ones((_RB, 1), jnp.float32), v[:, :-1]], axis=1)
    s = 1
    while s < _S:
        p = p * jnp.concatenate(
            [jnp.ones((_RB, s), jnp.float32), p[:, :-s]], axis=1)
        s *= 2
    weights = alpha * p

    rgb_ref[...] = jnp.sum(weights[:, :, None] * rgb_r, axis=1)
    depth_ref[...] = jnp.sum(weights * zvs, axis=1, keepdims=True)
    acc_ref[...] = jnp.sum(weights, axis=1, keepdims=True)
    sem_ref[...] = jnp.sum(weights[:, :, None] * (sem_r * onehot), axis=1)
    oh_ref[...] = onehot.reshape(_RB * _S, _NC)


def kernel(rays, intersection, W1, W2, Wsig, Wsem, Wfeat, Wr1, Wr2):
    f32 = jnp.float32
    rays2 = rays.reshape(_R, 6)
    inter2 = intersection.reshape(_R, _NB, 4)
    w1m = jnp.concatenate([W1[jnp.asarray(_PERM_POS)],
                           jnp.zeros((1, _WH), f32)], axis=0)
    wcat = jnp.concatenate([Wfeat, Wsem, Wsig], axis=1)
    wr1f = Wr1[:_WH]
    wr1d = jnp.concatenate([Wr1[_WH:][jnp.asarray(_PERM_DIR)],
                            jnp.zeros((5, _WH // 2), f32)], axis=0)

    nblk = _R // _RB
    out_shapes = (
        jax.ShapeDtypeStruct((_R, 3), f32),
        jax.ShapeDtypeStruct((_R, 1), f32),
        jax.ShapeDtypeStruct((_R, 1), f32),
        jax.ShapeDtypeStruct((_R, _NC), f32),
        jax.ShapeDtypeStruct((_R * _S, _NC), f32),
    )
    rgb_m, depth_m, acc_m, sem_m, oh = pl.pallas_call(
        _body,
        grid=(nblk,),
        in_specs=[
            pl.BlockSpec((_RB, 6), lambda i: (i, 0)),
            pl.BlockSpec((_RB, _NB, 4), lambda i: (i, 0, 0)),
            pl.BlockSpec((1, _NZ), lambda i: (0, 0)),
            pl.BlockSpec((1, _CASC), lambda i: (0, 0)),
            pl.BlockSpec((1, _CASC), lambda i: (0, 0)),
            pl.BlockSpec((1, _CASC), lambda i: (0, 0)),
            pl.BlockSpec((64, _WH), lambda i: (0, 0)),
            pl.BlockSpec((_WH, _WH), lambda i: (0, 0)),
            pl.BlockSpec((_WH, _WH + _NC + 1), lambda i: (0, 0)),
            pl.BlockSpec((_WH, _WH // 2), lambda i: (0, 0)),
            pl.BlockSpec((32, _WH // 2), lambda i: (0, 0)),
            pl.BlockSpec((_WH // 2, 3), lambda i: (0, 0)),
        ],
        out_specs=(
            pl.BlockSpec((_RB, 3), lambda i: (i, 0)),
            pl.BlockSpec((_RB, 1), lambda i: (i, 0)),
            pl.BlockSpec((_RB, 1), lambda i: (i, 0)),
            pl.BlockSpec((_RB, _NC), lambda i: (i, 0)),
            pl.BlockSpec((_RB * _S, _NC), lambda i: (i, 0)),
        ),
        out_shape=out_shapes,
    )(rays2, inter2,
      jnp.asarray(_TVEC).reshape(1, _NZ),
      jnp.asarray(_BELOW).reshape(1, _CASC),
      jnp.asarray(_ABOVE).reshape(1, _CASC),
      jnp.asarray(_TFRAC).reshape(1, _CASC),
      w1m, W2, wcat, wr1f, wr1d, Wr2)

    return (rgb_m.reshape(1, _R, 3),
            depth_m.reshape(1, _R),
            acc_m.reshape(1, _R),
            sem_m.reshape(1, _R, _NC),
            oh.reshape(1, _R * _S, _NC))


# speedup vs baseline: 8.0491x; 1.2692x over previous
import numpy as np
import jax
import jax.numpy as jnp
from jax.experimental import pallas as pl

_R = 1024
_NB = 8
_SP = 8
_NZ = _NB * _SP
_CASC = 128
_S = _CASC + 2 * _NB
_NC = 50
_DIST = 100.0
_FR_POS = 10
_FR_DIR = 4
_WH = 128
_MERGE_PAIRS = [(39, 41), (27, 26), (28, 26), (29, 26), (30, 26), (31, 26),
                (9, 8), (35, 13)]

_RB = 64

_MM = np.eye(_NC, dtype=np.float32)
for _i, _t in _MERGE_PAIRS:
    _MM[_i, _i] = 0.0
    _MM[_i, _t] = 1.0


def _pdf_consts():
    w = np.full((_NZ - 1,), 1.0, np.float32) + np.float32(1e-5)
    pdf = (w / w.sum(dtype=np.float32)).astype(np.float32)
    cdf = np.concatenate([np.zeros((1,), np.float32),
                          np.cumsum(pdf, dtype=np.float32).astype(np.float32)])
    u = np.linspace(0.0, 1.0, _CASC).astype(np.float32)
    inds = np.sum((u[:, None] >= cdf[None, :]).astype(np.int32), axis=-1)
    below = np.clip(inds - 1, 0, _NZ - 1).astype(np.int32)
    above = np.clip(inds, 0, _NZ - 1).astype(np.int32)
    denom = cdf[above] - cdf[below]
    denom = np.where(denom < 1e-5, np.float32(1.0), denom).astype(np.float32)
    t = ((u - cdf[below]) / denom).astype(np.float32)
    return below, above, t

_BELOW, _ABOVE, _TFRAC = _pdf_consts()
_TVEC = np.tile(np.linspace(0.0, 1.0, _SP).astype(np.float32), _NB)
_FREQS_POS = (2.0 ** np.arange(_FR_POS)).astype(np.float32)
_FREQS_DIR = (2.0 ** np.arange(_FR_DIR)).astype(np.float32)

def _enc_perm(L):
    return np.array(
        [0, 1, 2]
        + [3 + 6 * l + c for l in range(L) for c in range(3)]
        + [6 + 6 * l + c for l in range(L) for c in range(3)], np.int32)

_PERM_POS = _enc_perm(_FR_POS)
_PERM_DIR = _enc_perm(_FR_DIR)


def _ranks(vals):
    n = vals.shape[-1]
    ei = jax.lax.broadcasted_iota(jnp.int32, (n, n), 0)
    fi = jax.lax.broadcasted_iota(jnp.int32, (n, n), 1)
    a = vals[:, :, None]
    b = vals[:, None, :]
    less = (b < a) | ((b == a) & (fi < ei)[None])
    return jnp.sum(less.astype(jnp.int32), axis=2)


def _body(rays_ref, inter_ref, tvec_ref, below_ref, above_ref, tfrac_ref,
          w1_ref, w2_ref, wcat_ref, wr1f_ref, wr1d_ref,
          wr2_ref, mm_ref, rgb_ref, depth_ref, acc_ref, sem_ref, oh_ref):
    rays = rays_ref[...]
    origin = rays[:, 0:3]
    dvec = rays[:, 3:6]
    inter = inter_ref[...]
    near = inter[:, :, 0]
    far = inter[:, :, 1]
    cls = inter[:, :, 3].astype(jnp.int32)
    scale = jnp.sqrt(jnp.sum(dvec * dvec, axis=-1, keepdims=True))

    tvec = tvec_ref[...]
    grp = jax.lax.broadcasted_iota(jnp.int32, (1, _NZ), 1) // _SP
    near64 = jnp.zeros((_RB, _NZ), jnp.float32)
    far64 = jnp.zeros((_RB, _NZ), jnp.float32)
    for b in range(_NB):
        near64 = jnp.where(grp == b, near[:, b:b + 1], near64)
        far64 = jnp.where(grp == b, far[:, b:b + 1], far64)
    zf = near64 * (1.0 - tvec) + far64 * tvec

    rank64 = _ranks(zf)
    below = below_ref[...]
    above = above_ref[...]
    tfrac = tfrac_ref[...]
    m0 = rank64[:, :, None] == below[:, None, :]
    m1 = rank64[:, :, None] == above[:, None, :]
    bg0 = jnp.sum(jnp.where(m0, zf[:, :, None], 0.0), axis=1)
    bg1 = jnp.sum(jnp.where(m1, zf[:, :, None], 0.0), axis=1)
    zv128 = bg0 + tfrac * (bg1 - bg0)

    zb = jnp.concatenate([near - 1e-5, far + 1e-5], axis=1)
    ia = jax.lax.broadcasted_iota(jnp.int32, (1, _CASC), 1)
    cnt_b = jnp.sum((zb[:, None, :] < zv128[:, :, None]).astype(jnp.int32),
                    axis=2)
    rank_a = ia + cnt_b
    rank_b = jnp.sum((zv128[:, None, :] <= zb[:, :, None]).astype(jnp.int32),
                     axis=2) + _ranks(zb)
    k144a = jax.lax.broadcasted_iota(jnp.int32, (_CASC, _S), 1)
    k144b = jax.lax.broadcasted_iota(jnp.int32, (2 * _NB, _S), 1)
    ma = rank_a[:, :, None] == k144a[None]
    mb = rank_b[:, :, None] == k144b[None]
    zv = (jnp.sum(jnp.where(ma, zv128[:, :, None], 0.0), axis=1) +
          jnp.sum(jnp.where(mb, zb[:, :, None], 0.0), axis=1))

    pts = dvec[:, None, :] * zv[:, :, None] / scale[:, :, None]
    xyz = (origin[:, None, :] + pts) / _DIST
    x = xyz.reshape(_RB * _S, 3)
    xall = jnp.concatenate([x * f for f in _FREQS_POS], axis=1)
    pe = jnp.concatenate(
        [x, jnp.sin(xall), jnp.cos(xall),
         jnp.zeros((_RB * _S, 1), jnp.float32)], axis=1)
    dall = jnp.concatenate([dvec * f for f in _FREQS_DIR], axis=1)
    dpe = jnp.concatenate(
        [dvec, jnp.sin(dall), jnp.cos(dall),
         jnp.zeros((_RB, 5), jnp.float32)], axis=1)

    h = jnp.maximum(jnp.dot(pe, w1_ref[...],
                            preferred_element_type=jnp.float32), 0.0)
    h = jnp.maximum(jnp.dot(h, w2_ref[...],
                            preferred_element_type=jnp.float32), 0.0)
    hcat = jnp.dot(h, wcat_ref[...], preferred_element_type=jnp.float32)
    feat = hcat[:, :_WH]
    ddot = jnp.dot(dpe, wr1d_ref[...],
                   preferred_element_type=jnp.float32)
    ddot_b = jnp.broadcast_to(ddot[:, None, :], (_RB, _S, 64)).reshape(
        _RB * _S, 64)
    hr = jnp.maximum(jnp.dot(feat, wr1f_ref[...],
                             preferred_element_type=jnp.float32) + ddot_b, 0.0)
    rgbl = jnp.dot(hr, wr2_ref[...], preferred_element_type=jnp.float32)
    rgb = jax.nn.sigmoid(rgbl)

    sem_r = hcat[:, _WH:_WH + _NC].reshape(_RB, _S, _NC)
    sigma3 = hcat[:, _WH + _NC:_WH + _NC + 1].reshape(_RB, _S, 1)
    sigma_r = jnp.sum(sigma3, axis=2)
    rgb_r = rgb.reshape(_RB, _S, 3)

    inside = ((zv[:, :, None] > near[:, None, :]) &
              (zv[:, :, None] < far[:, None, :]))
    dfar = zv[:, :, None] - far[:, None, :]
    dnear = near[:, None, :] - zv[:, :, None]
    bound = (((dfar < 1e-3) & (dfar > 0)) |
             ((dnear > 0) & (dnear < 1e-3)))
    bound_any = jnp.sum(bound.astype(jnp.int32), axis=2) > 0

    c50 = jax.lax.broadcasted_iota(jnp.int32, (_RB, _NB, _NC), 2)
    ohc0 = (cls[:, :, None] == c50).astype(jnp.float32)
    ohc = jnp.dot(ohc0.reshape(_RB * _NB, _NC), mm_ref[...],
                  preferred_element_type=jnp.float32).reshape(_RB, _NB, _NC)
    inside_f = inside.astype(jnp.float32)
    onehot = jnp.zeros((_RB, _S, _NC), jnp.float32)
    for b in range(_NB):
        onehot = onehot + inside_f[:, :, b:b + 1] * ohc[:, b:b + 1, :]
    onehot = jnp.minimum(onehot, 1.0)

    ssum = jnp.sum(onehot, axis=2)
    ssum3 = jnp.sum(onehot, axis=2, keepdims=True)
    mask_bbox = (zv < _DIST) & (ssum == 0.0)
    zv3 = zv[:, :, None]
    mask_bbox3 = (zv3 < _DIST) & (ssum3 == 0.0)
    mask_bg3 = (zv3 > _DIST) & (ssum3 == 0.0)
    col = jax.lax.broadcasted_iota(jnp.int32, (_RB, _S, _NC), 2)
    onehot = jnp.where((col == 0) & mask_bbox3, 1.0, onehot)
    onehot = jnp.where((col == 23) & mask_bg3, 1.0, onehot)

    sigma_r = jnp.where(mask_bbox | bound_any, 0.0, sigma_r)
    zvs = zv / scale
    dists = jnp.concatenate(
        [zvs[:, 1:] - zvs[:, :-1],
         jnp.full((_RB, 1), 1e10, jnp.float32)], axis=1)
    dists = dists * scale
    alpha = 1.0 - jnp.exp(-jnp.maximum(sigma_r, 0.0) * dists)
    v = 1.0 - alpha + 1e-10
    p = jnp.concatenate([jnp.ones((_RB, 1), jnp.float32), v[:, :-1]], axis=1)
    s = 1
    while s < _S:
        p = p * jnp.concatenate(
            [jnp.ones((_RB, s), jnp.float32), p[:, :-s]], axis=1)
        s *= 2
    weights = alpha * p

    rgb_ref[...] = jnp.sum(weights[:, :, None] * rgb_r, axis=1)
    depth_ref[...] = jnp.sum(weights * zvs, axis=1, keepdims=True)
    acc_ref[...] = jnp.sum(weights, axis=1, keepdims=True)
    sem_ref[...] = jnp.sum(weights[:, :, None] * (sem_r * onehot), axis=1)
    oh_ref[...] = onehot.reshape(_RB * _S, _NC)


def kernel(rays, intersection, W1, W2, Wsig, Wsem, Wfeat, Wr1, Wr2):
    f32 = jnp.float32
    rays2 = rays.reshape(_R, 6)
    inter2 = intersection.reshape(_R, _NB, 4)
    w1m = jnp.concatenate([W1[jnp.asarray(_PERM_POS)],
                           jnp.zeros((1, _WH), f32)], axis=0)
    wcat = jnp.concatenate([Wfeat, Wsem, Wsig], axis=1)
    wr1f = Wr1[:_WH]
    wr1d = jnp.concatenate([Wr1[_WH:][jnp.asarray(_PERM_DIR)],
                            jnp.zeros((5, _WH // 2), f32)], axis=0)

    nblk = _R // _RB
    out_shapes = (
        jax.ShapeDtypeStruct((_R, 3), f32),
        jax.ShapeDtypeStruct((_R, 1), f32),
        jax.ShapeDtypeStruct((_R, 1), f32),
        jax.ShapeDtypeStruct((_R, _NC), f32),
        jax.ShapeDtypeStruct((_R * _S, _NC), f32),
    )
    rgb_m, depth_m, acc_m, sem_m, oh = pl.pallas_call(
        _body,
        grid=(nblk,),
        in_specs=[
            pl.BlockSpec((_RB, 6), lambda i: (i, 0)),
            pl.BlockSpec((_RB, _NB, 4), lambda i: (i, 0, 0)),
            pl.BlockSpec((1, _NZ), lambda i: (0, 0)),
            pl.BlockSpec((1, _CASC), lambda i: (0, 0)),
            pl.BlockSpec((1, _CASC), lambda i: (0, 0)),
            pl.BlockSpec((1, _CASC), lambda i: (0, 0)),
            pl.BlockSpec((64, _WH), lambda i: (0, 0)),
            pl.BlockSpec((_WH, _WH), lambda i: (0, 0)),
            pl.BlockSpec((_WH, _WH + _NC + 1), lambda i: (0, 0)),
            pl.BlockSpec((_WH, _WH // 2), lambda i: (0, 0)),
            pl.BlockSpec((32, _WH // 2), lambda i: (0, 0)),
            pl.BlockSpec((_WH // 2, 3), lambda i: (0, 0)),
            pl.BlockSpec((_NC, _NC), lambda i: (0, 0)),
        ],
        out_specs=(
            pl.BlockSpec((_RB, 3), lambda i: (i, 0)),
            pl.BlockSpec((_RB, 1), lambda i: (i, 0)),
            pl.BlockSpec((_RB, 1), lambda i: (i, 0)),
            pl.BlockSpec((_RB, _NC), lambda i: (i, 0)),
            pl.BlockSpec((_RB * _S, _NC), lambda i: (i, 0)),
        ),
        out_shape=out_shapes,
    )(rays2, inter2,
      jnp.asarray(_TVEC).reshape(1, _NZ),
      jnp.asarray(_BELOW).reshape(1, _CASC),
      jnp.asarray(_ABOVE).reshape(1, _CASC),
      jnp.asarray(_TFRAC).reshape(1, _CASC),
      w1m, W2, wcat, wr1f, wr1d, Wr2, jnp.asarray(_MM))

    return (rgb_m.reshape(1, _R, 3),
            depth_m.reshape(1, _R),
            acc_m.reshape(1, _R),
            sem_m.reshape(1, _R, _NC),
            oh.reshape(1, _R * _S, _NC))


# speedup vs baseline: 8.0497x; 1.0001x over previous
import numpy as np
import jax
import jax.numpy as jnp
from jax.experimental import pallas as pl
from jax.experimental.pallas import tpu as pltpu

_R = 1024
_NB = 8
_SP = 8
_NZ = _NB * _SP
_CASC = 128
_S = _CASC + 2 * _NB
_NC = 50
_DIST = 100.0
_FR_POS = 10
_FR_DIR = 4
_WH = 128
_MERGE_PAIRS = [(39, 41), (27, 26), (28, 26), (29, 26), (30, 26), (31, 26),
                (9, 8), (35, 13)]

_RB = 64

_MM = np.eye(_NC, dtype=np.float32)
for _i, _t in _MERGE_PAIRS:
    _MM[_i, _i] = 0.0
    _MM[_i, _t] = 1.0


def _pdf_consts():
    w = np.full((_NZ - 1,), 1.0, np.float32) + np.float32(1e-5)
    pdf = (w / w.sum(dtype=np.float32)).astype(np.float32)
    cdf = np.concatenate([np.zeros((1,), np.float32),
                          np.cumsum(pdf, dtype=np.float32).astype(np.float32)])
    u = np.linspace(0.0, 1.0, _CASC).astype(np.float32)
    inds = np.sum((u[:, None] >= cdf[None, :]).astype(np.int32), axis=-1)
    below = np.clip(inds - 1, 0, _NZ - 1).astype(np.int32)
    above = np.clip(inds, 0, _NZ - 1).astype(np.int32)
    denom = cdf[above] - cdf[below]
    denom = np.where(denom < 1e-5, np.float32(1.0), denom).astype(np.float32)
    t = ((u - cdf[below]) / denom).astype(np.float32)
    return below, above, t

_BELOW, _ABOVE, _TFRAC = _pdf_consts()
_TVEC = np.tile(np.linspace(0.0, 1.0, _SP).astype(np.float32), _NB)
_FREQS_POS = (2.0 ** np.arange(_FR_POS)).astype(np.float32)
_FREQS_DIR = (2.0 ** np.arange(_FR_DIR)).astype(np.float32)

def _enc_perm(L):
    return np.array(
        [0, 1, 2]
        + [3 + 6 * l + c for l in range(L) for c in range(3)]
        + [6 + 6 * l + c for l in range(L) for c in range(3)], np.int32)

_PERM_POS = _enc_perm(_FR_POS)
_PERM_DIR = _enc_perm(_FR_DIR)


def _ranks(vals):
    n = vals.shape[-1]
    ei = jax.lax.broadcasted_iota(jnp.int32, (n, n), 0)
    fi = jax.lax.broadcasted_iota(jnp.int32, (n, n), 1)
    a = vals[:, :, None]
    b = vals[:, None, :]
    less = (b < a) | ((b == a) & (fi < ei)[None])
    return jnp.sum(less.astype(jnp.int32), axis=2)


def _body(rays_ref, inter_ref, tvec_ref, below_ref, above_ref, tfrac_ref,
          w1_ref, w2_ref, wcat_ref, wr1f_ref, wr1d_ref,
          wr2_ref, mm_ref, rgb_ref, depth_ref, acc_ref, sem_ref, oh_ref):
    rays = rays_ref[...]
    origin = rays[:, 0:3]
    dvec = rays[:, 3:6]
    inter = inter_ref[...]
    near = inter[:, :, 0]
    far = inter[:, :, 1]
    cls = inter[:, :, 3].astype(jnp.int32)
    scale = jnp.sqrt(jnp.sum(dvec * dvec, axis=-1, keepdims=True))

    tvec = tvec_ref[...]
    grp = jax.lax.broadcasted_iota(jnp.int32, (1, _NZ), 1) // _SP
    near64 = jnp.zeros((_RB, _NZ), jnp.float32)
    far64 = jnp.zeros((_RB, _NZ), jnp.float32)
    for b in range(_NB):
        near64 = jnp.where(grp == b, near[:, b:b + 1], near64)
        far64 = jnp.where(grp == b, far[:, b:b + 1], far64)
    zf = near64 * (1.0 - tvec) + far64 * tvec

    rank64 = _ranks(zf)
    below = below_ref[...]
    above = above_ref[...]
    tfrac = tfrac_ref[...]
    m0 = rank64[:, :, None] == below[:, None, :]
    m1 = rank64[:, :, None] == above[:, None, :]
    bg0 = jnp.sum(jnp.where(m0, zf[:, :, None], 0.0), axis=1)
    bg1 = jnp.sum(jnp.where(m1, zf[:, :, None], 0.0), axis=1)
    zv128 = bg0 + tfrac * (bg1 - bg0)

    zb = jnp.concatenate([near - 1e-5, far + 1e-5], axis=1)
    ia = jax.lax.broadcasted_iota(jnp.int32, (1, _CASC), 1)
    cnt_b = jnp.sum((zb[:, None, :] < zv128[:, :, None]).astype(jnp.int32),
                    axis=2)
    rank_a = ia + cnt_b
    rank_b = jnp.sum((zv128[:, None, :] <= zb[:, :, None]).astype(jnp.int32),
                     axis=2) + _ranks(zb)
    k144a = jax.lax.broadcasted_iota(jnp.int32, (_CASC, _S), 1)
    k144b = jax.lax.broadcasted_iota(jnp.int32, (2 * _NB, _S), 1)
    ma = rank_a[:, :, None] == k144a[None]
    mb = rank_b[:, :, None] == k144b[None]
    zv = (jnp.sum(jnp.where(ma, zv128[:, :, None], 0.0), axis=1) +
          jnp.sum(jnp.where(mb, zb[:, :, None], 0.0), axis=1))

    pts = dvec[:, None, :] * zv[:, :, None] / scale[:, :, None]
    xyz = (origin[:, None, :] + pts) / _DIST
    x = xyz.reshape(_RB * _S, 3)
    xall = jnp.concatenate([x * f for f in _FREQS_POS], axis=1)
    pe = jnp.concatenate(
        [x, jnp.sin(xall), jnp.cos(xall),
         jnp.zeros((_RB * _S, 1), jnp.float32)], axis=1)
    dall = jnp.concatenate([dvec * f for f in _FREQS_DIR], axis=1)
    dpe = jnp.concatenate(
        [dvec, jnp.sin(dall), jnp.cos(dall),
         jnp.zeros((_RB, 5), jnp.float32)], axis=1)

    h = jnp.maximum(jnp.dot(pe, w1_ref[...],
                            preferred_element_type=jnp.float32), 0.0)
    h = jnp.maximum(jnp.dot(h, w2_ref[...],
                            preferred_element_type=jnp.float32), 0.0)
    hcat = jnp.dot(h, wcat_ref[...], preferred_element_type=jnp.float32)
    feat = hcat[:, :_WH]
    ddot = jnp.dot(dpe, wr1d_ref[...],
                   preferred_element_type=jnp.float32)
    ddot_b = jnp.broadcast_to(ddot[:, None, :], (_RB, _S, 64)).reshape(
        _RB * _S, 64)
    hr = jnp.maximum(jnp.dot(feat, wr1f_ref[...],
                             preferred_element_type=jnp.float32) + ddot_b, 0.0)
    rgbl = jnp.dot(hr, wr2_ref[...], preferred_element_type=jnp.float32)
    rgb = jax.nn.sigmoid(rgbl)

    sem_r = hcat[:, _WH:_WH + _NC].reshape(_RB, _S, _NC)
    sigma3 = hcat[:, _WH + _NC:_WH + _NC + 1].reshape(_RB, _S, 1)
    sigma_r = jnp.sum(sigma3, axis=2)
    rgb_r = rgb.reshape(_RB, _S, 3)

    inside = ((zv[:, :, None] > near[:, None, :]) &
              (zv[:, :, None] < far[:, None, :]))
    dfar = zv[:, :, None] - far[:, None, :]
    dnear = near[:, None, :] - zv[:, :, None]
    bound = (((dfar < 1e-3) & (dfar > 0)) |
             ((dnear > 0) & (dnear < 1e-3)))
    bound_any = jnp.sum(bound.astype(jnp.int32), axis=2) > 0

    c50 = jax.lax.broadcasted_iota(jnp.int32, (_RB, _NB, _NC), 2)
    ohc0 = (cls[:, :, None] == c50).astype(jnp.float32)
    ohc = jnp.dot(ohc0.reshape(_RB * _NB, _NC), mm_ref[...],
                  preferred_element_type=jnp.float32).reshape(_RB, _NB, _NC)
    inside_f = inside.astype(jnp.float32)
    onehot = jnp.zeros((_RB, _S, _NC), jnp.float32)
    for b in range(_NB):
        onehot = onehot + inside_f[:, :, b:b + 1] * ohc[:, b:b + 1, :]
    onehot = jnp.minimum(onehot, 1.0)

    ssum = jnp.sum(onehot, axis=2)
    ssum3 = jnp.sum(onehot, axis=2, keepdims=True)
    mask_bbox = (zv < _DIST) & (ssum == 0.0)
    zv3 = zv[:, :, None]
    mask_bbox3 = (zv3 < _DIST) & (ssum3 == 0.0)
    mask_bg3 = (zv3 > _DIST) & (ssum3 == 0.0)
    col = jax.lax.broadcasted_iota(jnp.int32, (_RB, _S, _NC), 2)
    onehot = jnp.where((col == 0) & mask_bbox3, 1.0, onehot)
    onehot = jnp.where((col == 23) & mask_bg3, 1.0, onehot)

    sigma_r = jnp.where(mask_bbox | bound_any, 0.0, sigma_r)
    zvs = zv / scale
    dists = jnp.concatenate(
        [zvs[:, 1:] - zvs[:, :-1],
         jnp.full((_RB, 1), 1e10, jnp.float32)], axis=1)
    dists = dists * scale
    alpha = 1.0 - jnp.exp(-jnp.maximum(sigma_r, 0.0) * dists)
    v = 1.0 - alpha + 1e-10
    p = jnp.concatenate([jnp.ones((_RB, 1), jnp.float32), v[:, :-1]], axis=1)
    s = 1
    while s < _S:
        p = p * jnp.concatenate(
            [jnp.ones((_RB, s), jnp.float32), p[:, :-s]], axis=1)
        s *= 2
    weights = alpha * p

    rgb_ref[...] = jnp.sum(weights[:, :, None] * rgb_r, axis=1)
    depth_ref[...] = jnp.sum(weights * zvs, axis=1, keepdims=True)
    acc_ref[...] = jnp.sum(weights, axis=1, keepdims=True)
    sem_ref[...] = jnp.sum(weights[:, :, None] * (sem_r * onehot), axis=1)
    oh_ref[...] = onehot.reshape(_RB * _S, _NC)


def kernel(rays, intersection, W1, W2, Wsig, Wsem, Wfeat, Wr1, Wr2):
    f32 = jnp.float32
    rays2 = rays.reshape(_R, 6)
    inter2 = intersection.reshape(_R, _NB, 4)
    w1m = jnp.concatenate([W1[jnp.asarray(_PERM_POS)],
                           jnp.zeros((1, _WH), f32)], axis=0)
    wcat = jnp.concatenate([Wfeat, Wsem, Wsig], axis=1)
    wr1f = Wr1[:_WH]
    wr1d = jnp.concatenate([Wr1[_WH:][jnp.asarray(_PERM_DIR)],
                            jnp.zeros((5, _WH // 2), f32)], axis=0)

    nblk = _R // _RB
    out_shapes = (
        jax.ShapeDtypeStruct((_R, 3), f32),
        jax.ShapeDtypeStruct((_R, 1), f32),
        jax.ShapeDtypeStruct((_R, 1), f32),
        jax.ShapeDtypeStruct((_R, _NC), f32),
        jax.ShapeDtypeStruct((_R * _S, _NC), f32),
    )
    rgb_m, depth_m, acc_m, sem_m, oh = pl.pallas_call(
        _body,
        grid=(nblk,),
        in_specs=[
            pl.BlockSpec((_RB, 6), lambda i: (i, 0)),
            pl.BlockSpec((_RB, _NB, 4), lambda i: (i, 0, 0)),
            pl.BlockSpec((1, _NZ), lambda i: (0, 0)),
            pl.BlockSpec((1, _CASC), lambda i: (0, 0)),
            pl.BlockSpec((1, _CASC), lambda i: (0, 0)),
            pl.BlockSpec((1, _CASC), lambda i: (0, 0)),
            pl.BlockSpec((64, _WH), lambda i: (0, 0)),
            pl.BlockSpec((_WH, _WH), lambda i: (0, 0)),
            pl.BlockSpec((_WH, _WH + _NC + 1), lambda i: (0, 0)),
            pl.BlockSpec((_WH, _WH // 2), lambda i: (0, 0)),
            pl.BlockSpec((32, _WH // 2), lambda i: (0, 0)),
            pl.BlockSpec((_WH // 2, 3), lambda i: (0, 0)),
            pl.BlockSpec((_NC, _NC), lambda i: (0, 0)),
        ],
        out_specs=(
            pl.BlockSpec((_RB, 3), lambda i: (i, 0)),
            pl.BlockSpec((_RB, 1), lambda i: (i, 0)),
            pl.BlockSpec((_RB, 1), lambda i: (i, 0)),
            pl.BlockSpec((_RB, _NC), lambda i: (i, 0)),
            pl.BlockSpec((_RB * _S, _NC), lambda i: (i, 0)),
        ),
        out_shape=out_shapes,
        compiler_params=pltpu.CompilerParams(
            dimension_semantics=("parallel",)),
    )(rays2, inter2,
      jnp.asarray(_TVEC).reshape(1, _NZ),
      jnp.asarray(_BELOW).reshape(1, _CASC),
      jnp.asarray(_ABOVE).reshape(1, _CASC),
      jnp.asarray(_TFRAC).reshape(1, _CASC),
      w1m, W2, wcat, wr1f, wr1d, Wr2, jnp.asarray(_MM))

    return (rgb_m.reshape(1, _R, 3),
            depth_m.reshape(1, _R),
            acc_m.reshape(1, _R),
            sem_m.reshape(1, _R, _NC),
            oh.reshape(1, _R * _S, _NC))


# speedup vs baseline: 9.3711x; 1.1641x over previous
import numpy as np
import jax
import jax.numpy as jnp
from jax.experimental import pallas as pl
from jax.experimental.pallas import tpu as pltpu

_R = 1024
_NB = 8
_SP = 8
_NZ = _NB * _SP
_CASC = 128
_S = _CASC + 2 * _NB
_NC = 50
_DIST = 100.0
_FR_POS = 10
_FR_DIR = 4
_WH = 128
_MERGE_PAIRS = [(39, 41), (27, 26), (28, 26), (29, 26), (30, 26), (31, 26),
                (9, 8), (35, 13)]

_RB = 64

_MM = np.eye(_NC, dtype=np.float32)
for _i, _t in _MERGE_PAIRS:
    _MM[_i, _i] = 0.0
    _MM[_i, _t] = 1.0


def _pdf_consts():
    w = np.full((_NZ - 1,), 1.0, np.float32) + np.float32(1e-5)
    pdf = (w / w.sum(dtype=np.float32)).astype(np.float32)
    cdf = np.concatenate([np.zeros((1,), np.float32),
                          np.cumsum(pdf, dtype=np.float32).astype(np.float32)])
    u = np.linspace(0.0, 1.0, _CASC).astype(np.float32)
    inds = np.sum((u[:, None] >= cdf[None, :]).astype(np.int32), axis=-1)
    below = np.clip(inds - 1, 0, _NZ - 1).astype(np.int32)
    above = np.clip(inds, 0, _NZ - 1).astype(np.int32)
    denom = cdf[above] - cdf[below]
    denom = np.where(denom < 1e-5, np.float32(1.0), denom).astype(np.float32)
    t = ((u - cdf[below]) / denom).astype(np.float32)
    return below, above, t

_BELOW, _ABOVE, _TFRAC = _pdf_consts()
_TVEC = np.tile(np.linspace(0.0, 1.0, _SP).astype(np.float32), _NB)
_FREQS_POS = (2.0 ** np.arange(_FR_POS)).astype(np.float32)
_FREQS_DIR = (2.0 ** np.arange(_FR_DIR)).astype(np.float32)

def _enc_perm(L):
    return np.array(
        [0, 1, 2]
        + [3 + 6 * l + c for l in range(L) for c in range(3)]
        + [6 + 6 * l + c for l in range(L) for c in range(3)], np.int32)

_PERM_POS = _enc_perm(_FR_POS)
_PERM_DIR = _enc_perm(_FR_DIR)

_ENC_CONSTS = np.zeros((4, 64), np.float32)
for _c in range(63):
    _coord = _c if _c < 3 else (_c - 3) % 3
    _ENC_CONSTS[_coord, _c] = 1.0
    if _c < 3:
        _ENC_CONSTS[3, _c] = 1.0
    elif _c < 33:
        _ENC_CONSTS[3, _c] = _FREQS_POS[(_c - 3) // 3]
    else:
        _ENC_CONSTS[3, _c] = _FREQS_POS[(_c - 33) // 3]


def _ranks(vals):
    n = vals.shape[-1]
    ei = jax.lax.broadcasted_iota(jnp.int32, (n, n), 0)
    fi = jax.lax.broadcasted_iota(jnp.int32, (n, n), 1)
    a = vals[:, :, None]
    b = vals[:, None, :]
    less = (b < a) | ((b == a) & (fi < ei)[None])
    return jnp.sum(less.astype(jnp.int32), axis=2)


def _body(rays_ref, inter_ref, tvec_ref, below_ref, above_ref, tfrac_ref,
          w1_ref, w2_ref, wcat_ref, wr1f_ref, wr1d_ref,
          wr2_ref, mm_ref, encc_ref,
          rgb_ref, depth_ref, acc_ref, sem_ref, oh_ref):
    rays = rays_ref[...]
    origin = rays[:, 0:3]
    dvec = rays[:, 3:6]
    inter = inter_ref[...]
    near = inter[:, :, 0]
    far = inter[:, :, 1]
    cls = inter[:, :, 3].astype(jnp.int32)
    scale = jnp.sqrt(jnp.sum(dvec * dvec, axis=-1, keepdims=True))

    tvec = tvec_ref[...]
    grp = jax.lax.broadcasted_iota(jnp.int32, (1, _NZ), 1) // _SP
    near64 = jnp.zeros((_RB, _NZ), jnp.float32)
    far64 = jnp.zeros((_RB, _NZ), jnp.float32)
    for b in range(_NB):
        near64 = jnp.where(grp == b, near[:, b:b + 1], near64)
        far64 = jnp.where(grp == b, far[:, b:b + 1], far64)
    zf = near64 * (1.0 - tvec) + far64 * tvec

    rank64 = _ranks(zf)
    below = below_ref[...]
    above = above_ref[...]
    tfrac = tfrac_ref[...]
    m0 = rank64[:, :, None] == below[:, None, :]
    m1 = rank64[:, :, None] == above[:, None, :]
    bg0 = jnp.sum(jnp.where(m0, zf[:, :, None], 0.0), axis=1)
    bg1 = jnp.sum(jnp.where(m1, zf[:, :, None], 0.0), axis=1)
    zv128 = bg0 + tfrac * (bg1 - bg0)

    zb = jnp.concatenate([near - 1e-5, far + 1e-5], axis=1)
    ia = jax.lax.broadcasted_iota(jnp.int32, (1, _CASC), 1)
    cnt_b = jnp.sum((zb[:, None, :] < zv128[:, :, None]).astype(jnp.int32),
                    axis=2)
    rank_a = ia + cnt_b
    rank_b = jnp.sum((zv128[:, None, :] <= zb[:, :, None]).astype(jnp.int32),
                     axis=2) + _ranks(zb)
    k144a = jax.lax.broadcasted_iota(jnp.int32, (_CASC, _S), 1)
    k144b = jax.lax.broadcasted_iota(jnp.int32, (2 * _NB, _S), 1)
    ma = rank_a[:, :, None] == k144a[None]
    mb = rank_b[:, :, None] == k144b[None]
    zv = (jnp.sum(jnp.where(ma, zv128[:, :, None], 0.0), axis=1) +
          jnp.sum(jnp.where(mb, zb[:, :, None], 0.0), axis=1))

    pts = dvec[:, None, :] * zv[:, :, None] / scale[:, :, None]
    xyz = (origin[:, None, :] + pts) / _DIST
    enc = encc_ref[...]
    mrep0 = enc[None, 0:1, :]
    mrep1 = enc[None, 1:2, :]
    mrep2 = enc[None, 2:3, :]
    fmul = enc[None, 3:4, :]
    xrep = (xyz[:, :, 0:1] * mrep0 + xyz[:, :, 1:2] * mrep1 +
            xyz[:, :, 2:3] * mrep2)
    args = xrep * fmul
    col64 = jax.lax.broadcasted_iota(jnp.int32, (1, 1, 64), 2)
    pe3 = jnp.where(col64 < 3, xrep,
                    jnp.where(col64 < 33, jnp.sin(args), jnp.cos(args)))
    pe = pe3.reshape(_RB * _S, 64)
    dall = jnp.concatenate([dvec * f for f in _FREQS_DIR], axis=1)
    dpe = jnp.concatenate(
        [dvec, jnp.sin(dall), jnp.cos(dall),
         jnp.zeros((_RB, 5), jnp.float32)], axis=1)

    h = jnp.maximum(jnp.dot(pe, w1_ref[...],
                            preferred_element_type=jnp.float32), 0.0)
    h = jnp.maximum(jnp.dot(h, w2_ref[...],
                            preferred_element_type=jnp.float32), 0.0)
    hcat = jnp.dot(h, wcat_ref[...], preferred_element_type=jnp.float32)
    feat = hcat[:, :_WH]
    ddot = jnp.dot(dpe, wr1d_ref[...],
                   preferred_element_type=jnp.float32)
    ddot_b = jnp.broadcast_to(ddot[:, None, :], (_RB, _S, 64)).reshape(
        _RB * _S, 64)
    hr = jnp.maximum(jnp.dot(feat, wr1f_ref[...],
                             preferred_element_type=jnp.float32) + ddot_b, 0.0)
    rgbl = jnp.dot(hr, wr2_ref[...], preferred_element_type=jnp.float32)
    rgb = jax.nn.sigmoid(rgbl)

    sem_r = hcat[:, _WH:_WH + _NC].reshape(_RB, _S, _NC)
    sigma3 = hcat[:, _WH + _NC:_WH + _NC + 1].reshape(_RB, _S, 1)
    sigma_r = jnp.sum(sigma3, axis=2)
    rgb_r = rgb.reshape(_RB, _S, 3)

    inside = ((zv[:, :, None] > near[:, None, :]) &
              (zv[:, :, None] < far[:, None, :]))
    dfar = zv[:, :, None] - far[:, None, :]
    dnear = near[:, None, :] - zv[:, :, None]
    bound = (((dfar < 1e-3) & (dfar > 0)) |
             ((dnear > 0) & (dnear < 1e-3)))
    bound_any = jnp.sum(bound.astype(jnp.int32), axis=2) > 0

    c50 = jax.lax.broadcasted_iota(jnp.int32, (_RB, _NB, _NC), 2)
    ohc0 = (cls[:, :, None] == c50).astype(jnp.float32)
    ohc = jnp.dot(ohc0.reshape(_RB * _NB, _NC), mm_ref[...],
                  preferred_element_type=jnp.float32).reshape(_RB, _NB, _NC)
    inside_f = inside.astype(jnp.float32)
    onehot = jnp.zeros((_RB, _S, _NC), jnp.float32)
    for b in range(_NB):
        onehot = onehot + inside_f[:, :, b:b + 1] * ohc[:, b:b + 1, :]
    onehot = jnp.minimum(onehot, 1.0)

    ssum = jnp.sum(onehot, axis=2)
    ssum3 = jnp.sum(onehot, axis=2, keepdims=True)
    mask_bbox = (zv < _DIST) & (ssum == 0.0)
    zv3 = zv[:, :, None]
    mask_bbox3 = (zv3 < _DIST) & (ssum3 == 0.0)
    mask_bg3 = (zv3 > _DIST) & (ssum3 == 0.0)
    col = jax.lax.broadcasted_iota(jnp.int32, (_RB, _S, _NC), 2)
    onehot = jnp.where((col == 0) & mask_bbox3, 1.0, onehot)
    onehot = jnp.where((col == 23) & mask_bg3, 1.0, onehot)

    sigma_r = jnp.where(mask_bbox | bound_any, 0.0, sigma_r)
    zvs = zv / scale
    dists = jnp.concatenate(
        [zvs[:, 1:] - zvs[:, :-1],
         jnp.full((_RB, 1), 1e10, jnp.float32)], axis=1)
    dists = dists * scale
    alpha = 1.0 - jnp.exp(-jnp.maximum(sigma_r, 0.0) * dists)
    v = 1.0 - alpha + 1e-10
    p = jnp.concatenate([jnp.ones((_RB, 1), jnp.float32), v[:, :-1]], axis=1)
    s = 1
    while s < _S:
        p = p * jnp.concatenate(
            [jnp.ones((_RB, s), jnp.float32), p[:, :-s]], axis=1)
        s *= 2
    weights = alpha * p

    rgb_ref[...] = jnp.sum(weights[:, :, None] * rgb_r, axis=1)
    depth_ref[...] = jnp.sum(weights * zvs, axis=1, keepdims=True)
    acc_ref[...] = jnp.sum(weights, axis=1, keepdims=True)
    sem_ref[...] = jnp.sum(weights[:, :, None] * (sem_r * onehot), axis=1)
    oh_ref[...] = onehot.reshape(_RB * _S, _NC)


def kernel(rays, intersection, W1, W2, Wsig, Wsem, Wfeat, Wr1, Wr2):
    f32 = jnp.float32
    rays2 = rays.reshape(_R, 6)
    inter2 = intersection.reshape(_R, _NB, 4)
    w1m = jnp.concatenate([W1[jnp.asarray(_PERM_POS)],
                           jnp.zeros((1, _WH), f32)], axis=0)
    wcat = jnp.concatenate([Wfeat, Wsem, Wsig], axis=1)
    wr1f = Wr1[:_WH]
    wr1d = jnp.concatenate([Wr1[_WH:][jnp.asarray(_PERM_DIR)],
                            jnp.zeros((5, _WH // 2), f32)], axis=0)

    nblk = _R // _RB
    out_shapes = (
        jax.ShapeDtypeStruct((_R, 3), f32),
        jax.ShapeDtypeStruct((_R, 1), f32),
        jax.ShapeDtypeStruct((_R, 1), f32),
        jax.ShapeDtypeStruct((_R, _NC), f32),
        jax.ShapeDtypeStruct((_R * _S, _NC), f32),
    )
    rgb_m, depth_m, acc_m, sem_m, oh = pl.pallas_call(
        _body,
        grid=(nblk,),
        in_specs=[
            pl.BlockSpec((_RB, 6), lambda i: (i, 0)),
            pl.BlockSpec((_RB, _NB, 4), lambda i: (i, 0, 0)),
            pl.BlockSpec((1, _NZ), lambda i: (0, 0)),
            pl.BlockSpec((1, _CASC), lambda i: (0, 0)),
            pl.BlockSpec((1, _CASC), lambda i: (0, 0)),
            pl.BlockSpec((1, _CASC), lambda i: (0, 0)),
            pl.BlockSpec((64, _WH), lambda i: (0, 0)),
            pl.BlockSpec((_WH, _WH), lambda i: (0, 0)),
            pl.BlockSpec((_WH, _WH + _NC + 1), lambda i: (0, 0)),
            pl.BlockSpec((_WH, _WH // 2), lambda i: (0, 0)),
            pl.BlockSpec((32, _WH // 2), lambda i: (0, 0)),
            pl.BlockSpec((_WH // 2, 3), lambda i: (0, 0)),
            pl.BlockSpec((_NC, _NC), lambda i: (0, 0)),
            pl.BlockSpec((4, 64), lambda i: (0, 0)),
        ],
        out_specs=(
            pl.BlockSpec((_RB, 3), lambda i: (i, 0)),
            pl.BlockSpec((_RB, 1), lambda i: (i, 0)),
            pl.BlockSpec((_RB, 1), lambda i: (i, 0)),
            pl.BlockSpec((_RB, _NC), lambda i: (i, 0)),
            pl.BlockSpec((_RB * _S, _NC), lambda i: (i, 0)),
        ),
        out_shape=out_shapes,
        compiler_params=pltpu.CompilerParams(
            dimension_semantics=("parallel",)),
    )(rays2, inter2,
      jnp.asarray(_TVEC).reshape(1, _NZ),
      jnp.asarray(_BELOW).reshape(1, _CASC),
      jnp.asarray(_ABOVE).reshape(1, _CASC),
      jnp.asarray(_TFRAC).reshape(1, _CASC),
      w1m, W2, wcat, wr1f, wr1d, Wr2, jnp.asarray(_MM),
      jnp.asarray(_ENC_CONSTS))

    return (rgb_m.reshape(1, _R, 3),
            depth_m.reshape(1, _R),
            acc_m.reshape(1, _R),
            sem_m.reshape(1, _R, _NC),
            oh.reshape(1, _R * _S, _NC))


# speedup vs baseline: 9.3769x; 1.0006x over previous
import numpy as np
import jax
import jax.numpy as jnp
from jax.experimental import pallas as pl
from jax.experimental.pallas import tpu as pltpu

_R = 1024
_NB = 8
_SP = 8
_NZ = _NB * _SP
_CASC = 128
_S = _CASC + 2 * _NB
_NC = 50
_DIST = 100.0
_FR_POS = 10
_FR_DIR = 4
_WH = 128
_MERGE_PAIRS = [(39, 41), (27, 26), (28, 26), (29, 26), (30, 26), (31, 26),
                (9, 8), (35, 13)]

_RB = 64

_MM = np.eye(_NC, dtype=np.float32)
for _i, _t in _MERGE_PAIRS:
    _MM[_i, _i] = 0.0
    _MM[_i, _t] = 1.0


def _pdf_consts():
    w = np.full((_NZ - 1,), 1.0, np.float32) + np.float32(1e-5)
    pdf = (w / w.sum(dtype=np.float32)).astype(np.float32)
    cdf = np.concatenate([np.zeros((1,), np.float32),
                          np.cumsum(pdf, dtype=np.float32).astype(np.float32)])
    u = np.linspace(0.0, 1.0, _CASC).astype(np.float32)
    inds = np.sum((u[:, None] >= cdf[None, :]).astype(np.int32), axis=-1)
    below = np.clip(inds - 1, 0, _NZ - 1).astype(np.int32)
    above = np.clip(inds, 0, _NZ - 1).astype(np.int32)
    denom = cdf[above] - cdf[below]
    denom = np.where(denom < 1e-5, np.float32(1.0), denom).astype(np.float32)
    t = ((u - cdf[below]) / denom).astype(np.float32)
    return below, above, t

_BELOW, _ABOVE, _TFRAC = _pdf_consts()
_TVEC = np.tile(np.linspace(0.0, 1.0, _SP).astype(np.float32), _NB)
_FREQS_POS = (2.0 ** np.arange(_FR_POS)).astype(np.float32)
_FREQS_DIR = (2.0 ** np.arange(_FR_DIR)).astype(np.float32)

def _enc_consts(L, width):
    e = np.zeros((8, width), np.float32)
    for c in range(3 + 6 * L):
        if c < 3:
            coord, kind, f = c, 0, 1.0
        else:
            l, r = (c - 3) // 6, (c - 3) % 6
            coord = r % 3
            kind = 1 if r < 3 else 2
            f = float(2.0 ** l)
        e[coord, c] = 1.0
        e[3, c] = f
        e[4, c] = 1.0 if kind == 0 else 0.0
        e[5, c] = 1.0 if kind == 1 else 0.0
    return e

_ENC_POS = _enc_consts(_FR_POS, 64)
_ENC_DIR = _enc_consts(_FR_DIR, 32)


def _ranks(vals):
    n = vals.shape[-1]
    ei = jax.lax.broadcasted_iota(jnp.int32, (n, n), 0)
    fi = jax.lax.broadcasted_iota(jnp.int32, (n, n), 1)
    a = vals[:, :, None]
    b = vals[:, None, :]
    less = (b < a) | ((b == a) & (fi < ei)[None])
    return jnp.sum(less.astype(jnp.int32), axis=2)


def _body(rays_ref, inter_ref, tvec_ref, below_ref, above_ref, tfrac_ref,
          w1_ref, w2_ref, wcat_ref, wr1f_ref, wr1d_ref,
          wr2_ref, mm_ref, encc_ref, dencc_ref,
          rgb_ref, depth_ref, acc_ref, sem_ref, oh_ref):
    rays = rays_ref[...]
    origin = rays[:, 0:3]
    dvec = rays[:, 3:6]
    inter = inter_ref[...]
    near = inter[:, :, 0]
    far = inter[:, :, 1]
    cls = inter[:, :, 3].astype(jnp.int32)
    scale = jnp.sqrt(jnp.sum(dvec * dvec, axis=-1, keepdims=True))

    tvec = tvec_ref[...]
    grp = jax.lax.broadcasted_iota(jnp.int32, (1, _NZ), 1) // _SP
    near64 = jnp.zeros((_RB, _NZ), jnp.float32)
    far64 = jnp.zeros((_RB, _NZ), jnp.float32)
    for b in range(_NB):
        near64 = jnp.where(grp == b, near[:, b:b + 1], near64)
        far64 = jnp.where(grp == b, far[:, b:b + 1], far64)
    zf = near64 * (1.0 - tvec) + far64 * tvec

    rank64 = _ranks(zf)
    below = below_ref[...]
    above = above_ref[...]
    tfrac = tfrac_ref[...]
    m0 = rank64[:, :, None] == below[:, None, :]
    m1 = rank64[:, :, None] == above[:, None, :]
    bg0 = jnp.sum(jnp.where(m0, zf[:, :, None], 0.0), axis=1)
    bg1 = jnp.sum(jnp.where(m1, zf[:, :, None], 0.0), axis=1)
    zv128 = bg0 + tfrac * (bg1 - bg0)

    zb = jnp.concatenate([near - 1e-5, far + 1e-5], axis=1)
    ia = jax.lax.broadcasted_iota(jnp.int32, (1, _CASC), 1)
    cnt_b = jnp.sum((zb[:, None, :] < zv128[:, :, None]).astype(jnp.int32),
                    axis=2)
    rank_a = ia + cnt_b
    rank_b = jnp.sum((zv128[:, None, :] <= zb[:, :, None]).astype(jnp.int32),
                     axis=2) + _ranks(zb)
    k144a = jax.lax.broadcasted_iota(jnp.int32, (_CASC, _S), 1)
    k144b = jax.lax.broadcasted_iota(jnp.int32, (2 * _NB, _S), 1)
    ma = rank_a[:, :, None] == k144a[None]
    mb = rank_b[:, :, None] == k144b[None]
    zv = (jnp.sum(jnp.where(ma, zv128[:, :, None], 0.0), axis=1) +
          jnp.sum(jnp.where(mb, zb[:, :, None], 0.0), axis=1))

    pts = dvec[:, None, :] * zv[:, :, None] / scale[:, :, None]
    xyz = (origin[:, None, :] + pts) / _DIST
    enc = encc_ref[...]
    xrep = (xyz[:, :, 0:1] * enc[None, 0:1, :] +
            xyz[:, :, 1:2] * enc[None, 1:2, :] +
            xyz[:, :, 2:3] * enc[None, 2:3, :])
    args = xrep * enc[None, 3:4, :]
    pe3 = jnp.where(enc[None, 4:5, :] == 1.0, xrep,
                    jnp.where(enc[None, 5:6, :] == 1.0,
                              jnp.sin(args), jnp.cos(args)))
    pe = pe3.reshape(_RB * _S, 64)
    denc = dencc_ref[...]
    drep = (dvec[:, 0:1] * denc[0:1, :] + dvec[:, 1:2] * denc[1:2, :] +
            dvec[:, 2:3] * denc[2:3, :])
    dargs = drep * denc[3:4, :]
    dpe = jnp.where(denc[4:5, :] == 1.0, drep,
                    jnp.where(denc[5:6, :] == 1.0,
                              jnp.sin(dargs), jnp.cos(dargs)))

    h = jnp.maximum(jnp.dot(pe, w1_ref[...],
                            preferred_element_type=jnp.float32), 0.0)
    h = jnp.maximum(jnp.dot(h, w2_ref[...],
                            preferred_element_type=jnp.float32), 0.0)
    hcat = jnp.dot(h, wcat_ref[...], preferred_element_type=jnp.float32)
    feat = hcat[:, :_WH]
    ddot = jnp.dot(dpe, wr1d_ref[...],
                   preferred_element_type=jnp.float32)
    ddot_b = jnp.broadcast_to(ddot[:, None, :], (_RB, _S, 64)).reshape(
        _RB * _S, 64)
    hr = jnp.maximum(jnp.dot(feat, wr1f_ref[...],
                             preferred_element_type=jnp.float32) + ddot_b, 0.0)
    rgbl = jnp.dot(hr, wr2_ref[...], preferred_element_type=jnp.float32)
    rgb = jax.nn.sigmoid(rgbl)

    sem_r = hcat[:, _WH:_WH + _NC].reshape(_RB, _S, _NC)
    sigma3 = hcat[:, _WH + _NC:_WH + _NC + 1].reshape(_RB, _S, 1)
    sigma_r = jnp.sum(sigma3, axis=2)
    rgb_r = rgb.reshape(_RB, _S, 3)

    inside = ((zv[:, :, None] > near[:, None, :]) &
              (zv[:, :, None] < far[:, None, :]))
    dfar = zv[:, :, None] - far[:, None, :]
    dnear = near[:, None, :] - zv[:, :, None]
    bound = (((dfar < 1e-3) & (dfar > 0)) |
             ((dnear > 0) & (dnear < 1e-3)))
    bound_any = jnp.sum(bound.astype(jnp.int32), axis=2) > 0

    c50 = jax.lax.broadcasted_iota(jnp.int32, (_RB, _NB, _NC), 2)
    ohc0 = (cls[:, :, None] == c50).astype(jnp.float32)
    ohc = jnp.dot(ohc0.reshape(_RB * _NB, _NC), mm_ref[...],
                  preferred_element_type=jnp.float32).reshape(_RB, _NB, _NC)
    inside_f = inside.astype(jnp.float32)
    onehot = jnp.zeros((_RB, _S, _NC), jnp.float32)
    for b in range(_NB):
        onehot = onehot + inside_f[:, :, b:b + 1] * ohc[:, b:b + 1, :]
    onehot = jnp.minimum(onehot, 1.0)

    ssum = jnp.sum(onehot, axis=2)
    ssum3 = jnp.sum(onehot, axis=2, keepdims=True)
    mask_bbox = (zv < _DIST) & (ssum == 0.0)
    zv3 = zv[:, :, None]
    mask_bbox3 = (zv3 < _DIST) & (ssum3 == 0.0)
    mask_bg3 = (zv3 > _DIST) & (ssum3 == 0.0)
    col = jax.lax.broadcasted_iota(jnp.int32, (_RB, _S, _NC), 2)
    onehot = jnp.where((col == 0) & mask_bbox3, 1.0, onehot)
    onehot = jnp.where((col == 23) & mask_bg3, 1.0, onehot)

    sigma_r = jnp.where(mask_bbox | bound_any, 0.0, sigma_r)
    zvs = zv / scale
    dists = jnp.concatenate(
        [zvs[:, 1:] - zvs[:, :-1],
         jnp.full((_RB, 1), 1e10, jnp.float32)], axis=1)
    dists = dists * scale
    alpha = 1.0 - jnp.exp(-jnp.maximum(sigma_r, 0.0) * dists)
    v = 1.0 - alpha + 1e-10
    p = jnp.concatenate([jnp.ones((_RB, 1), jnp.float32), v[:, :-1]], axis=1)
    s = 1
    while s < _S:
        p = p * jnp.concatenate(
            [jnp.ones((_RB, s), jnp.float32), p[:, :-s]], axis=1)
        s *= 2
    weights = alpha * p

    rgb_ref[...] = jnp.sum(weights[:, :, None] * rgb_r, axis=1)
    depth_ref[...] = jnp.sum(weights * zvs, axis=1, keepdims=True)
    acc_ref[...] = jnp.sum(weights, axis=1, keepdims=True)
    sem_ref[...] = jnp.sum(weights[:, :, None] * (sem_r * onehot), axis=1)
    oh_ref[...] = onehot.reshape(_RB * _S, _NC)


def kernel(rays, intersection, W1, W2, Wsig, Wsem, Wfeat, Wr1, Wr2):
    f32 = jnp.float32
    rays2 = rays.reshape(_R, 6)
    inter2 = intersection.reshape(_R, _NB, 4)
    w1m = jnp.concatenate([W1, jnp.zeros((1, _WH), f32)], axis=0)
    wcat = jnp.concatenate([Wfeat, Wsem, Wsig], axis=1)
    wr1f = Wr1[:_WH]
    wr1d = jnp.concatenate([Wr1[_WH:],
                            jnp.zeros((5, _WH // 2), f32)], axis=0)

    nblk = _R // _RB
    out_shapes = (
        jax.ShapeDtypeStruct((_R, 3), f32),
        jax.ShapeDtypeStruct((_R, 1), f32),
        jax.ShapeDtypeStruct((_R, 1), f32),
        jax.ShapeDtypeStruct((_R, _NC), f32),
        jax.ShapeDtypeStruct((_R * _S, _NC), f32),
    )
    rgb_m, depth_m, acc_m, sem_m, oh = pl.pallas_call(
        _body,
        grid=(nblk,),
        in_specs=[
            pl.BlockSpec((_RB, 6), lambda i: (i, 0)),
            pl.BlockSpec((_RB, _NB, 4), lambda i: (i, 0, 0)),
            pl.BlockSpec((1, _NZ), lambda i: (0, 0)),
            pl.BlockSpec((1, _CASC), lambda i: (0, 0)),
            pl.BlockSpec((1, _CASC), lambda i: (0, 0)),
            pl.BlockSpec((1, _CASC), lambda i: (0, 0)),
            pl.BlockSpec((64, _WH), lambda i: (0, 0)),
            pl.BlockSpec((_WH, _WH), lambda i: (0, 0)),
            pl.BlockSpec((_WH, _WH + _NC + 1), lambda i: (0, 0)),
            pl.BlockSpec((_WH, _WH // 2), lambda i: (0, 0)),
            pl.BlockSpec((32, _WH // 2), lambda i: (0, 0)),
            pl.BlockSpec((_WH // 2, 3), lambda i: (0, 0)),
            pl.BlockSpec((_NC, _NC), lambda i: (0, 0)),
            pl.BlockSpec((8, 64), lambda i: (0, 0)),
            pl.BlockSpec((8, 32), lambda i: (0, 0)),
        ],
        out_specs=(
            pl.BlockSpec((_RB, 3), lambda i: (i, 0)),
            pl.BlockSpec((_RB, 1), lambda i: (i, 0)),
            pl.BlockSpec((_RB, 1), lambda i: (i, 0)),
            pl.BlockSpec((_RB, _NC), lambda i: (i, 0)),
            pl.BlockSpec((_RB * _S, _NC), lambda i: (i, 0)),
        ),
        out_shape=out_shapes,
        compiler_params=pltpu.CompilerParams(
            dimension_semantics=("parallel",)),
    )(rays2, inter2,
      jnp.asarray(_TVEC).reshape(1, _NZ),
      jnp.asarray(_BELOW).reshape(1, _CASC),
      jnp.asarray(_ABOVE).reshape(1, _CASC),
      jnp.asarray(_TFRAC).reshape(1, _CASC),
      w1m, W2, wcat, wr1f, wr1d, Wr2, jnp.asarray(_MM),
      jnp.asarray(_ENC_POS), jnp.asarray(_ENC_DIR))

    return (rgb_m.reshape(1, _R, 3),
            depth_m.reshape(1, _R),
            acc_m.reshape(1, _R),
            sem_m.reshape(1, _R, _NC),
            oh.reshape(1, _R * _S, _NC))
